# baseline (speedup 1.0000x reference)
import functools

import jax
import jax.numpy as jnp
from jax import lax
from jax.experimental import pallas as pl
from jax.experimental.pallas import tpu as pltpu
from jax.experimental.pallas import tpu_sc as plsc

N = 10000
E = 320000
F = 128
NPH = 4
FQ = F // NPH

NC = 2
NS = 16
NW = NC * NS
CHUNK = 128
NCHUNK = 80
EPW = NCHUNK * CHUNK
EPAD = EPW * NW
TCHUNK = EPAD // CHUNK
NPAD = 10240
STRIPE = NPAD // NS

RB = 1280
GRID = NPAD // RB

DEGW = 16


@functools.cache
def _mesh():
    return plsc.VectorSubcoreMesh(
        core_axis_name="c", subcore_axis_name="s", num_cores=NC, num_subcores=NS
    )


NBUF = 4
CPB = NCHUNK // NBUF


def _agg_body(g_hbm, srci, dsti, out_hbm,
              srcv, dstv, rows, gtab, acc, semg, sems):
    c = lax.axis_index("c")
    s = lax.axis_index("s")
    w = s * NC + c
    pltpu.sync_copy(srci.at[pl.ds(w * NCHUNK, NCHUNK)], srcv)
    pltpu.sync_copy(dsti.at[pl.ds(w * NCHUNK, NCHUNK)], dstv)
    stripe = pl.ds(s * STRIPE, STRIPE)

    for p in range(NPH):
        cols = pl.ds(p * FQ, FQ)
        pltpu.sync_copy(g_hbm.at[stripe, cols], gtab.at[stripe])
        pltpu.sync_copy(g_hbm.at[stripe, cols], acc.at[stripe])
        plsc.subcore_barrier()

        def gat(j, b):
            return pltpu.make_async_copy(gtab.at[srcv.at[j]], rows.at[b],
                                         semg.at[b])

        def sct(j, b):
            return pltpu.make_async_copy(rows.at[b], acc.at[dstv.at[j]],
                                         sems.at[b])

        for b in range(NBUF):
            gat(b, b).start()
        for b in range(NBUF):
            gat(b, b).wait()
            sct(b, b).start(add=True)

        def turn(k, carry):
            for b in range(NBUF):
                j = NBUF * k + b
                sct(j - NBUF, b).wait()
                gat(j, b).start()
            for b in range(NBUF):
                j = NBUF * k + b
                gat(j, b).wait()
                sct(j, b).start(add=True)
            return carry

        lax.fori_loop(1, CPB, turn, 0)
        for b in range(NBUF):
            sct(NCHUNK - NBUF + b, b).wait()

        plsc.subcore_barrier()
        pltpu.sync_copy(acc.at[stripe], out_hbm.at[c, stripe, cols])


@functools.cache
def _agg_call():
    return pl.kernel(
        _agg_body,
        out_type=jax.ShapeDtypeStruct((NC, NPAD, F), jnp.float32),
        name="sc_edge_agg",
        mesh=_mesh(),
        compiler_params=pltpu.CompilerParams(use_tc_tiling_on_sc=False),
        scratch_types=[
            pltpu.VMEM((NCHUNK, CHUNK), jnp.int32),
            pltpu.VMEM((NCHUNK, CHUNK), jnp.int32),
            pltpu.VMEM((NBUF, CHUNK, FQ), jnp.float32),
            pltpu.VMEM_SHARED((NPAD, FQ), jnp.float32),
            pltpu.VMEM_SHARED((NPAD, FQ), jnp.float32),
            pltpu.SemaphoreType.DMA((NBUF,)),
            pltpu.SemaphoreType.DMA((NBUF,)),
        ],
    )


def _deg_body(dsti, ones_hbm, zero_hbm, out_hbm, dstv, onesv, acc):
    c = lax.axis_index("c")
    s = lax.axis_index("s")
    w = s * NC + c
    pltpu.sync_copy(dsti.at[pl.ds(w * NCHUNK, NCHUNK)], dstv)
    pltpu.sync_copy(ones_hbm.at[w], onesv)
    stripe = pl.ds(s * STRIPE, STRIPE)
    pltpu.sync_copy(zero_hbm.at[w], acc.at[stripe])
    plsc.subcore_barrier()

    def body(j, carry):
        pltpu.sync_copy(onesv, acc.at[dstv.at[j]], add=True)
        return carry

    lax.fori_loop(0, NCHUNK, body, 0)
    plsc.subcore_barrier()
    pltpu.sync_copy(acc.at[stripe], out_hbm.at[c, stripe])


@functools.cache
def _deg_call():
    return pl.kernel(
        _deg_body,
        out_type=jax.ShapeDtypeStruct((NC, NPAD, DEGW), jnp.float32),
        name="sc_degree",
        mesh=_mesh(),
        compiler_params=pltpu.CompilerParams(use_tc_tiling_on_sc=False),
        scratch_types=[
            pltpu.VMEM((NCHUNK, CHUNK), jnp.int32),
            pltpu.VMEM((CHUNK, DEGW), jnp.float32),
            pltpu.VMEM_SHARED((NPAD, DEGW), jnp.float32),
        ],
    )


def _tc_g1_body(degp_ref, x_ref, w_ref, g_ref, dinv_ref):
    deg = degp_ref[0, :, 0:1] + degp_ref[1, :, 0:1] + 1.0
    dinv = lax.rsqrt(deg)
    h = jnp.dot(x_ref[...], w_ref[...], preferred_element_type=jnp.float32)
    g_ref[...] = h * dinv
    dinv_ref[...] = dinv


def _tc_mid_body(p_ref, gin_ref, dinv_ref, b_ref, w_ref, g_ref):
    dinv = dinv_ref[...]
    agg = p_ref[0] + p_ref[1] - gin_ref[...]
    t = jnp.maximum(dinv * agg + b_ref[...], 0.0)
    g_ref[...] = jnp.dot(t, w_ref[...], preferred_element_type=jnp.float32) * dinv


def _tc_out_body(p_ref, gin_ref, dinv_ref, b_ref, w_ref, bout_ref, y_ref):
    dinv = dinv_ref[...]
    agg = p_ref[0] + p_ref[1] - gin_ref[...]
    t = jnp.maximum(dinv * agg + b_ref[...], 0.0)
    logits = jnp.dot(t, w_ref[...], preferred_element_type=jnp.float32)
    logits = logits + bout_ref[...]
    m = jnp.max(logits, axis=1, keepdims=True)
    e = jnp.exp(logits - m)
    y_ref[...] = e / jnp.sum(e, axis=1, keepdims=True)


def _tc_g1(degp, xp, W1):
    return pl.pallas_call(
        _tc_g1_body,
        grid=(GRID,),
        in_specs=[
            pl.BlockSpec((NC, RB, DEGW), lambda i: (0, i, 0)),
            pl.BlockSpec((RB, F), lambda i: (i, 0)),
            pl.BlockSpec((F, F), lambda i: (0, 0)),
        ],
        out_specs=[
            pl.BlockSpec((RB, F), lambda i: (i, 0)),
            pl.BlockSpec((RB, 1), lambda i: (i, 0)),
        ],
        out_shape=[
            jax.ShapeDtypeStruct((NPAD, F), jnp.float32),
            jax.ShapeDtypeStruct((NPAD, 1), jnp.float32),
        ],
    )(degp, xp, W1)


def _tc_mid(p, gin, dinv, b, W):
    return pl.pallas_call(
        _tc_mid_body,
        grid=(GRID,),
        in_specs=[
            pl.BlockSpec((NC, RB, F), lambda i: (0, i, 0)),
            pl.BlockSpec((RB, F), lambda i: (i, 0)),
            pl.BlockSpec((RB, 1), lambda i: (i, 0)),
            pl.BlockSpec((1, F), lambda i: (0, 0)),
            pl.BlockSpec((F, F), lambda i: (0, 0)),
        ],
        out_specs=pl.BlockSpec((RB, F), lambda i: (i, 0)),
        out_shape=jax.ShapeDtypeStruct((NPAD, F), jnp.float32),
    )(p, gin, dinv, b, W)


def _tc_out(p, gin, dinv, b, W, bout):
    return pl.pallas_call(
        _tc_out_body,
        grid=(GRID,),
        in_specs=[
            pl.BlockSpec((NC, RB, F), lambda i: (0, i, 0)),
            pl.BlockSpec((RB, F), lambda i: (i, 0)),
            pl.BlockSpec((RB, 1), lambda i: (i, 0)),
            pl.BlockSpec((1, F), lambda i: (0, 0)),
            pl.BlockSpec((F, F), lambda i: (0, 0)),
            pl.BlockSpec((1, F), lambda i: (0, 0)),
        ],
        out_specs=pl.BlockSpec((RB, F), lambda i: (i, 0)),
        out_shape=jax.ShapeDtypeStruct((NPAD, F), jnp.float32),
    )(p, gin, dinv, b, W, bout)


def kernel(x, edge_index, batch, W1, b1, W2, b2, Wout, bout):
    f32 = jnp.float32
    src = edge_index[0]
    dst = edge_index[1]
    padlen = EPAD - E
    pad_idx = jnp.full((padlen,), N, jnp.int32)
    srcp = jnp.concatenate([src, pad_idx]).reshape(TCHUNK, CHUNK)
    dstp = jnp.concatenate([dst, pad_idx]).reshape(TCHUNK, CHUNK)
    xp = jnp.zeros((NPAD, F), f32).at[:N].set(x)
    zero_col = jnp.zeros((NW, STRIPE, DEGW), f32)
    ones_col = jnp.ones((NW, CHUNK, DEGW), f32)
    b1r = b1.reshape(1, F)
    b2r = b2.reshape(1, F)
    boutr = bout.reshape(1, F)

    degp = _deg_call()(dstp, ones_col, zero_col)
    g1, dinv = _tc_g1(degp, xp, W1)
    p1 = _agg_call()(g1, srcp, dstp)
    g2 = _tc_mid(p1, g1, dinv, b1r, W2)
    p2 = _agg_call()(g2, srcp, dstp)
    y = _tc_out(p2, g2, dinv, b2r, Wout, boutr)
    return y[:N]


# speedup vs baseline: 1.1022x; 1.1022x over previous
import functools

import jax
import jax.numpy as jnp
from jax import lax
from jax.experimental import pallas as pl
from jax.experimental.pallas import tpu as pltpu
from jax.experimental.pallas import tpu_sc as plsc

N = 10000
E = 320000
F = 128
NPH = 4
FQ = F // NPH

NC = 2
NS = 16
NW = NC * NS
CHUNK = 128
NCHUNK = 80
EPW = NCHUNK * CHUNK
EPAD = EPW * NW
TCHUNK = EPAD // CHUNK
NPAD = 10240
STRIPE = NPAD // NS

RB = 1280
GRID = NPAD // RB

DEGW = 16


@functools.cache
def _mesh():
    return plsc.VectorSubcoreMesh(
        core_axis_name="c", subcore_axis_name="s", num_cores=NC, num_subcores=NS
    )


def _agg_body(g_hbm, srci, dsti, out_hbm,
              srcv, dstv, rows0, rows1, gtab, acc, sem0, sem1):
    c = lax.axis_index("c")
    s = lax.axis_index("s")
    w = s * NC + c
    pltpu.sync_copy(srci.at[pl.ds(w * NCHUNK, NCHUNK)], srcv)
    pltpu.sync_copy(dsti.at[pl.ds(w * NCHUNK, NCHUNK)], dstv)
    stripe = pl.ds(s * STRIPE, STRIPE)

    for p in range(NPH):
        cols = pl.ds(p * FQ, FQ)
        pltpu.sync_copy(g_hbm.at[stripe, cols], gtab.at[stripe])
        pltpu.sync_copy(g_hbm.at[stripe, cols], acc.at[stripe])
        plsc.subcore_barrier()

        pltpu.async_copy(gtab.at[srcv.at[0]], rows0, sem0)

        def pair(j, carry):
            pltpu.make_async_copy(gtab.at[srcv.at[j]], rows0, sem0).wait()
            pltpu.async_copy(gtab.at[srcv.at[j + 1]], rows1, sem1)
            pltpu.sync_copy(rows0, acc.at[dstv.at[j]], add=True)
            pltpu.make_async_copy(gtab.at[srcv.at[j + 1]], rows1, sem1).wait()
            nxt = lax.rem(j + 2, NCHUNK)
            pltpu.async_copy(gtab.at[srcv.at[nxt]], rows0, sem0)
            pltpu.sync_copy(rows1, acc.at[dstv.at[j + 1]], add=True)
            return carry

        lax.fori_loop(0, NCHUNK // 2, lambda i, v: pair(2 * i, v), 0)
        pltpu.make_async_copy(gtab.at[srcv.at[0]], rows0, sem0).wait()

        plsc.subcore_barrier()
        pltpu.sync_copy(acc.at[stripe], out_hbm.at[c, stripe, cols])


@functools.cache
def _agg_call():
    return pl.kernel(
        _agg_body,
        out_type=jax.ShapeDtypeStruct((NC, NPAD, F), jnp.float32),
        name="sc_edge_agg",
        mesh=_mesh(),
        compiler_params=pltpu.CompilerParams(use_tc_tiling_on_sc=False),
        scratch_types=[
            pltpu.VMEM((NCHUNK, CHUNK), jnp.int32),
            pltpu.VMEM((NCHUNK, CHUNK), jnp.int32),
            pltpu.VMEM((CHUNK, FQ), jnp.float32),
            pltpu.VMEM((CHUNK, FQ), jnp.float32),
            pltpu.VMEM_SHARED((NPAD, FQ), jnp.float32),
            pltpu.VMEM_SHARED((NPAD, FQ), jnp.float32),
            pltpu.SemaphoreType.DMA,
            pltpu.SemaphoreType.DMA,
        ],
    )


def _deg_body(dsti, ones_hbm, zero_hbm, out_hbm, dstv, onesv, acc):
    c = lax.axis_index("c")
    s = lax.axis_index("s")
    w = s * NC + c
    pltpu.sync_copy(dsti.at[pl.ds(w * NCHUNK, NCHUNK)], dstv)
    pltpu.sync_copy(ones_hbm.at[w], onesv)
    stripe = pl.ds(s * STRIPE, STRIPE)
    pltpu.sync_copy(zero_hbm.at[w], acc.at[stripe])
    plsc.subcore_barrier()

    def body(j, carry):
        pltpu.sync_copy(onesv, acc.at[dstv.at[j]], add=True)
        return carry

    lax.fori_loop(0, NCHUNK, body, 0)
    plsc.subcore_barrier()
    pltpu.sync_copy(acc.at[stripe], out_hbm.at[c, stripe])


@functools.cache
def _deg_call():
    return pl.kernel(
        _deg_body,
        out_type=jax.ShapeDtypeStruct((NC, NPAD, DEGW), jnp.float32),
        name="sc_degree",
        mesh=_mesh(),
        compiler_params=pltpu.CompilerParams(use_tc_tiling_on_sc=False),
        scratch_types=[
            pltpu.VMEM((NCHUNK, CHUNK), jnp.int32),
            pltpu.VMEM((CHUNK, DEGW), jnp.float32),
            pltpu.VMEM_SHARED((NPAD, DEGW), jnp.float32),
        ],
    )


def _tc_g1_body(degp_ref, x_ref, w_ref, g_ref, dinv_ref):
    deg = degp_ref[0, :, 0:1] + degp_ref[1, :, 0:1] + 1.0
    dinv = lax.rsqrt(deg)
    h = jnp.dot(x_ref[...], w_ref[...], preferred_element_type=jnp.float32)
    g_ref[...] = h * dinv
    dinv_ref[...] = dinv


def _tc_mid_body(p_ref, gin_ref, dinv_ref, b_ref, w_ref, g_ref):
    dinv = dinv_ref[...]
    agg = p_ref[0] + p_ref[1] - gin_ref[...]
    t = jnp.maximum(dinv * agg + b_ref[...], 0.0)
    g_ref[...] = jnp.dot(t, w_ref[...], preferred_element_type=jnp.float32) * dinv


def _tc_out_body(p_ref, gin_ref, dinv_ref, b_ref, w_ref, bout_ref, y_ref):
    dinv = dinv_ref[...]
    agg = p_ref[0] + p_ref[1] - gin_ref[...]
    t = jnp.maximum(dinv * agg + b_ref[...], 0.0)
    logits = jnp.dot(t, w_ref[...], preferred_element_type=jnp.float32)
    logits = logits + bout_ref[...]
    m = jnp.max(logits, axis=1, keepdims=True)
    e = jnp.exp(logits - m)
    y_ref[...] = e / jnp.sum(e, axis=1, keepdims=True)


def _tc_g1(degp, xp, W1):
    return pl.pallas_call(
        _tc_g1_body,
        grid=(GRID,),
        in_specs=[
            pl.BlockSpec((NC, RB, DEGW), lambda i: (0, i, 0)),
            pl.BlockSpec((RB, F), lambda i: (i, 0)),
            pl.BlockSpec((F, F), lambda i: (0, 0)),
        ],
        out_specs=[
            pl.BlockSpec((RB, F), lambda i: (i, 0)),
            pl.BlockSpec((RB, 1), lambda i: (i, 0)),
        ],
        out_shape=[
            jax.ShapeDtypeStruct((NPAD, F), jnp.float32),
            jax.ShapeDtypeStruct((NPAD, 1), jnp.float32),
        ],
    )(degp, xp, W1)


def _tc_mid(p, gin, dinv, b, W):
    return pl.pallas_call(
        _tc_mid_body,
        grid=(GRID,),
        in_specs=[
            pl.BlockSpec((NC, RB, F), lambda i: (0, i, 0)),
            pl.BlockSpec((RB, F), lambda i: (i, 0)),
            pl.BlockSpec((RB, 1), lambda i: (i, 0)),
            pl.BlockSpec((1, F), lambda i: (0, 0)),
            pl.BlockSpec((F, F), lambda i: (0, 0)),
        ],
        out_specs=pl.BlockSpec((RB, F), lambda i: (i, 0)),
        out_shape=jax.ShapeDtypeStruct((NPAD, F), jnp.float32),
    )(p, gin, dinv, b, W)


def _tc_out(p, gin, dinv, b, W, bout):
    return pl.pallas_call(
        _tc_out_body,
        grid=(GRID,),
        in_specs=[
            pl.BlockSpec((NC, RB, F), lambda i: (0, i, 0)),
            pl.BlockSpec((RB, F), lambda i: (i, 0)),
            pl.BlockSpec((RB, 1), lambda i: (i, 0)),
            pl.BlockSpec((1, F), lambda i: (0, 0)),
            pl.BlockSpec((F, F), lambda i: (0, 0)),
            pl.BlockSpec((1, F), lambda i: (0, 0)),
        ],
        out_specs=pl.BlockSpec((RB, F), lambda i: (i, 0)),
        out_shape=jax.ShapeDtypeStruct((NPAD, F), jnp.float32),
    )(p, gin, dinv, b, W, bout)


def kernel(x, edge_index, batch, W1, b1, W2, b2, Wout, bout):
    f32 = jnp.float32
    src = edge_index[0]
    dst = edge_index[1]
    padlen = EPAD - E
    pad_idx = jnp.full((padlen,), N, jnp.int32)
    srcp = jnp.concatenate([src, pad_idx]).reshape(TCHUNK, CHUNK)
    dstp = jnp.concatenate([dst, pad_idx]).reshape(TCHUNK, CHUNK)
    xp = jnp.zeros((NPAD, F), f32).at[:N].set(x)
    zero_col = jnp.zeros((NW, STRIPE, DEGW), f32)
    ones_col = jnp.ones((NW, CHUNK, DEGW), f32)
    b1r = b1.reshape(1, F)
    b2r = b2.reshape(1, F)
    boutr = bout.reshape(1, F)

    degp = _deg_call()(dstp, ones_col, zero_col)
    g1, dinv = _tc_g1(degp, xp, W1)
    p1 = _agg_call()(g1, srcp, dstp)
    g2 = _tc_mid(p1, g1, dinv, b1r, W2)
    p2 = _agg_call()(g2, srcp, dstp)
    y = _tc_out(p2, g2, dinv, b2r, Wout, boutr)
    return y[:N]
